# baseline (speedup 1.0000x reference)
import jax
import jax.numpy as jnp
from jax import lax
from jax.experimental import pallas as pl
from jax.experimental.pallas import tpu as pltpu
from jax.experimental.pallas import tpu_sc as plsc

_ALPHA = 1.0
_BETA = 0.2
_DEPTH = 3
_BASETHETA = 1.0

N = 8192
BLOCK_ROWS = 256

R_SC = 1024
R_TC = N - R_SC

NUM_WORKERS = 32
LANES = 16
SPAN = R_SC * N // NUM_WORKERS
CHUNK = 16384
N_CHUNKS = SPAN // CHUNK


def _jacobi_coeffs(L):
    A_l = (2 * L + _ALPHA + _BETA) * (2 * L + _ALPHA + _BETA - 1) / (
        2 * L * (L + _ALPHA + _BETA))
    B_l = (2 * L + _ALPHA + _BETA - 1) * (_ALPHA ** 2 - _BETA ** 2) / (
        2 * L * (L + _ALPHA + _BETA) * (2 * L + _ALPHA + _BETA - 2))
    C_l = (L + _ALPHA - 1) * (L + _BETA - 1) * (2 * L + _ALPHA + _BETA) / (
        L * (L + _ALPHA + _BETA) * (2 * L + _ALPHA + _BETA - 2))
    return A_l, B_l, C_l


def _cubic_coeffs(th):
    p = 0.5 * (_ALPHA - _BETA)
    q = 0.5 * (_ALPHA + _BETA + 2.0)
    A2, B2, C2 = _jacobi_coeffs(2)
    A3, B3, C3 = _jacobi_coeffs(3)
    x2_0 = B2 * p - C2
    x2_1 = A2 * p + B2 * q
    x2_2 = A2 * q
    x3_0 = B3 * x2_0 - C3 * p
    x3_1 = A3 * x2_0 + B3 * x2_1 - C3 * q
    x3_2 = A3 * x2_1 + B3 * x2_2
    x3_3 = A3 * x2_2
    c0 = th[0] + th[1] * p + th[2] * x2_0 + th[3] * x3_0
    c1 = th[1] * q + th[2] * x2_1 + th[3] * x3_1
    c2 = th[2] * x2_2 + th[3] * x3_2
    c3 = th[3] * x3_3
    return c0, c1, c2, c3


def _tc_body(adj_ref, th_ref, out_ref):
    a = adj_ref[...]
    th = _BASETHETA * jnp.tanh(th_ref[0, :])
    c0, c1, c2, c3 = _cubic_coeffs(th)
    out_ref[...] = c0 + a * (c1 + a * (c2 + a * c3))


def _tc_part(adj, th2d):
    grid = (R_TC // BLOCK_ROWS,)
    row0 = R_SC // BLOCK_ROWS
    return pl.pallas_call(
        _tc_body,
        grid=grid,
        in_specs=[
            pl.BlockSpec((BLOCK_ROWS, N), lambda i: (i + row0, 0)),
            pl.BlockSpec((1, _DEPTH + 1), lambda i: (0, 0)),
        ],
        out_specs=pl.BlockSpec((BLOCK_ROWS, N), lambda i: (i, 0)),
        out_shape=jax.ShapeDtypeStruct((R_TC, N), jnp.float32),
        compiler_params=pltpu.CompilerParams(
            dimension_semantics=("arbitrary",),
        ),
    )(adj, th2d)


def _sc_body(adj_hbm, coef_hbm, out_hbm, coef_v, in_v, out_v):
    wid = lax.axis_index("s") * 2 + lax.axis_index("c")
    pltpu.sync_copy(coef_hbm, coef_v)
    c0 = coef_v[pl.ds(0, LANES)]
    c1 = coef_v[pl.ds(LANES, LANES)]
    c2 = coef_v[pl.ds(2 * LANES, LANES)]
    c3 = coef_v[pl.ds(3 * LANES, LANES)]
    base0 = wid * SPAN

    def chunk_step(k, carry):
        base = base0 + k * CHUNK
        pltpu.sync_copy(adj_hbm.at[pl.ds(base, CHUNK)], in_v)

        def vec_step(i, c):
            a = in_v[pl.ds(i * LANES, LANES)]
            out_v[pl.ds(i * LANES, LANES)] = (
                c0 + a * (c1 + a * (c2 + a * c3)))
            return c

        lax.fori_loop(0, CHUNK // LANES, vec_step, 0)
        pltpu.sync_copy(out_v, out_hbm.at[pl.ds(base, CHUNK)])
        return carry

    lax.fori_loop(0, N_CHUNKS, chunk_step, 0)


def _sc_part(adj_flat, coef):
    mesh = plsc.VectorSubcoreMesh(core_axis_name="c", subcore_axis_name="s")
    run = pl.kernel(
        _sc_body,
        mesh=mesh,
        out_type=jax.ShapeDtypeStruct((R_SC * N,), jnp.float32),
        scratch_types=[
            pltpu.VMEM((4 * LANES,), jnp.float32),
            pltpu.VMEM((CHUNK,), jnp.float32),
            pltpu.VMEM((CHUNK,), jnp.float32),
        ],
    )
    return run(adj_flat, coef)


def kernel(adj, thetas):
    th = _BASETHETA * jnp.tanh(thetas)
    c0, c1, c2, c3 = _cubic_coeffs(th)
    coef = jnp.concatenate([
        jnp.full((LANES,), c, dtype=jnp.float32) for c in (c0, c1, c2, c3)
    ])
    th2d = thetas.reshape(1, _DEPTH + 1)

    sc_out = _sc_part(adj.reshape(-1), coef)
    tc_out = _tc_part(adj, th2d)
    return jnp.concatenate([sc_out.reshape(R_SC, N), tc_out], axis=0)


# speedup vs baseline: 1.4063x; 1.4063x over previous
import jax
import jax.numpy as jnp
from jax import lax
from jax.experimental import pallas as pl
from jax.experimental.pallas import tpu as pltpu
from jax.experimental.pallas import tpu_sc as plsc

_ALPHA = 1.0
_BETA = 0.2
_DEPTH = 3
_BASETHETA = 1.0

N = 8192
BLOCK_ROWS = 256

R_SC = 1024
R_TC = N - R_SC

NUM_WORKERS = 32
LANES = 16
SPAN = R_SC * N // NUM_WORKERS
CHUNK = 16384
N_CHUNKS = SPAN // CHUNK


def _jacobi_coeffs(L):
    A_l = (2 * L + _ALPHA + _BETA) * (2 * L + _ALPHA + _BETA - 1) / (
        2 * L * (L + _ALPHA + _BETA))
    B_l = (2 * L + _ALPHA + _BETA - 1) * (_ALPHA ** 2 - _BETA ** 2) / (
        2 * L * (L + _ALPHA + _BETA) * (2 * L + _ALPHA + _BETA - 2))
    C_l = (L + _ALPHA - 1) * (L + _BETA - 1) * (2 * L + _ALPHA + _BETA) / (
        L * (L + _ALPHA + _BETA) * (2 * L + _ALPHA + _BETA - 2))
    return A_l, B_l, C_l


def _cubic_coeffs(th):
    p = 0.5 * (_ALPHA - _BETA)
    q = 0.5 * (_ALPHA + _BETA + 2.0)
    A2, B2, C2 = _jacobi_coeffs(2)
    A3, B3, C3 = _jacobi_coeffs(3)
    x2_0 = B2 * p - C2
    x2_1 = A2 * p + B2 * q
    x2_2 = A2 * q
    x3_0 = B3 * x2_0 - C3 * p
    x3_1 = A3 * x2_0 + B3 * x2_1 - C3 * q
    x3_2 = A3 * x2_1 + B3 * x2_2
    x3_3 = A3 * x2_2
    c0 = th[0] + th[1] * p + th[2] * x2_0 + th[3] * x3_0
    c1 = th[1] * q + th[2] * x2_1 + th[3] * x3_1
    c2 = th[2] * x2_2 + th[3] * x3_2
    c3 = th[3] * x3_3
    return c0, c1, c2, c3


def _tc_body(adj_ref, th_ref, out_ref):
    a = adj_ref[...]
    th = _BASETHETA * jnp.tanh(th_ref[0, :])
    c0, c1, c2, c3 = _cubic_coeffs(th)
    out_ref[...] = c0 + a * (c1 + a * (c2 + a * c3))


def _tc_part(adj, th2d):
    grid = (R_TC // BLOCK_ROWS,)
    row0 = R_SC // BLOCK_ROWS
    return pl.pallas_call(
        _tc_body,
        grid=grid,
        in_specs=[
            pl.BlockSpec((BLOCK_ROWS, N), lambda i: (i + row0, 0)),
            pl.BlockSpec((1, _DEPTH + 1), lambda i: (0, 0)),
        ],
        out_specs=pl.BlockSpec((BLOCK_ROWS, N), lambda i: (i, 0)),
        out_shape=jax.ShapeDtypeStruct((R_TC, N), jnp.float32),
        compiler_params=pltpu.CompilerParams(
            dimension_semantics=("arbitrary",),
        ),
    )(adj, th2d)


ROWS_PER_WORKER = R_SC // NUM_WORKERS
BAND = 8
BANDS_PER_WORKER = ROWS_PER_WORKER // BAND


def _sc_body(adj_hbm, coef_hbm, out_hbm, coef_v, buf):
    wid = lax.axis_index("s") * 2 + lax.axis_index("c")
    pltpu.sync_copy(coef_hbm, coef_v)
    c0 = coef_v[pl.ds(0, LANES)]
    c1 = coef_v[pl.ds(LANES, LANES)]
    c2 = coef_v[pl.ds(2 * LANES, LANES)]
    c3 = coef_v[pl.ds(3 * LANES, LANES)]
    row0 = wid * ROWS_PER_WORKER

    def band_step(k, carry):
        r0 = row0 + k * BAND
        pltpu.sync_copy(adj_hbm.at[pl.ds(r0, BAND), :], buf)

        for r in range(BAND):
            def vec_step(i, c, r=r):
                a = buf[r, pl.ds(i * LANES, LANES)]
                buf[r, pl.ds(i * LANES, LANES)] = (
                    c0 + a * (c1 + a * (c2 + a * c3)))
                return c

            lax.fori_loop(0, N // LANES, vec_step, 0)

        pltpu.sync_copy(buf, out_hbm.at[pl.ds(r0, BAND), :])
        return carry

    lax.fori_loop(0, BANDS_PER_WORKER, band_step, 0)


def _sc_part(adj, coef):
    mesh = plsc.VectorSubcoreMesh(core_axis_name="c", subcore_axis_name="s")
    run = pl.kernel(
        _sc_body,
        mesh=mesh,
        out_type=jax.ShapeDtypeStruct((R_SC, N), jnp.float32),
        scratch_types=[
            pltpu.VMEM((4 * LANES,), jnp.float32),
            pltpu.VMEM((BAND, N), jnp.float32),
        ],
    )
    return run(adj, coef)


def kernel(adj, thetas):
    th = _BASETHETA * jnp.tanh(thetas)
    c0, c1, c2, c3 = _cubic_coeffs(th)
    coef = jnp.concatenate([
        jnp.full((LANES,), c, dtype=jnp.float32) for c in (c0, c1, c2, c3)
    ])
    th2d = thetas.reshape(1, _DEPTH + 1)

    sc_out = _sc_part(adj, coef)
    tc_out = _tc_part(adj, th2d)
    return jnp.concatenate([sc_out, tc_out], axis=0)


# speedup vs baseline: 3.3233x; 2.3631x over previous
import jax
import jax.numpy as jnp
from jax.experimental import pallas as pl
from jax.experimental.pallas import tpu as pltpu

_ALPHA = 1.0
_BETA = 0.2
_DEPTH = 3
_BASETHETA = 1.0

N = 8192
BLOCK_ROWS = 256


def _jacobi_coeffs(L):
    A_l = (2 * L + _ALPHA + _BETA) * (2 * L + _ALPHA + _BETA - 1) / (
        2 * L * (L + _ALPHA + _BETA))
    B_l = (2 * L + _ALPHA + _BETA - 1) * (_ALPHA ** 2 - _BETA ** 2) / (
        2 * L * (L + _ALPHA + _BETA) * (2 * L + _ALPHA + _BETA - 2))
    C_l = (L + _ALPHA - 1) * (L + _BETA - 1) * (2 * L + _ALPHA + _BETA) / (
        L * (L + _ALPHA + _BETA) * (2 * L + _ALPHA + _BETA - 2))
    return A_l, B_l, C_l


def _cubic_coeffs(th):
    p = 0.5 * (_ALPHA - _BETA)
    q = 0.5 * (_ALPHA + _BETA + 2.0)
    A2, B2, C2 = _jacobi_coeffs(2)
    A3, B3, C3 = _jacobi_coeffs(3)
    x2_0 = B2 * p - C2
    x2_1 = A2 * p + B2 * q
    x2_2 = A2 * q
    x3_0 = B3 * x2_0 - C3 * p
    x3_1 = A3 * x2_0 + B3 * x2_1 - C3 * q
    x3_2 = A3 * x2_1 + B3 * x2_2
    x3_3 = A3 * x2_2
    c0 = th[0] + th[1] * p + th[2] * x2_0 + th[3] * x3_0
    c1 = th[1] * q + th[2] * x2_1 + th[3] * x3_1
    c2 = th[2] * x2_2 + th[3] * x3_2
    c3 = th[3] * x3_3
    return c0, c1, c2, c3


def _poly_body(adj_ref, th_ref, out_ref):
    a = adj_ref[...]
    th = _BASETHETA * jnp.tanh(th_ref[0, :])
    c0, c1, c2, c3 = _cubic_coeffs(th)
    out_ref[...] = c0 + a * (c1 + a * (c2 + a * c3))


def kernel(adj, thetas):
    th2d = thetas.reshape(1, _DEPTH + 1)
    grid = (N // BLOCK_ROWS,)
    return pl.pallas_call(
        _poly_body,
        grid=grid,
        in_specs=[
            pl.BlockSpec((BLOCK_ROWS, N), lambda i: (i, 0)),
            pl.BlockSpec((1, _DEPTH + 1), lambda i: (0, 0)),
        ],
        out_specs=pl.BlockSpec((BLOCK_ROWS, N), lambda i: (i, 0)),
        out_shape=jax.ShapeDtypeStruct((N, N), jnp.float32),
        compiler_params=pltpu.CompilerParams(
            dimension_semantics=("parallel",),

        ),
    )(adj, th2d)


# speedup vs baseline: 3.4901x; 1.0502x over previous
import jax
import jax.numpy as jnp
from jax import lax
from jax.experimental import pallas as pl
from jax.experimental.pallas import tpu as pltpu

_ALPHA = 1.0
_BETA = 0.2
_DEPTH = 3
_BASETHETA = 1.0

N = 8192
BR = 128
NBUF = 4
NSTEPS = N // BR
ROUNDS = NSTEPS // NBUF


def _jacobi_coeffs(L):
    A_l = (2 * L + _ALPHA + _BETA) * (2 * L + _ALPHA + _BETA - 1) / (
        2 * L * (L + _ALPHA + _BETA))
    B_l = (2 * L + _ALPHA + _BETA - 1) * (_ALPHA ** 2 - _BETA ** 2) / (
        2 * L * (L + _ALPHA + _BETA) * (2 * L + _ALPHA + _BETA - 2))
    C_l = (L + _ALPHA - 1) * (L + _BETA - 1) * (2 * L + _ALPHA + _BETA) / (
        L * (L + _ALPHA + _BETA) * (2 * L + _ALPHA + _BETA - 2))
    return A_l, B_l, C_l


def _cubic_coeffs(th):
    p = 0.5 * (_ALPHA - _BETA)
    q = 0.5 * (_ALPHA + _BETA + 2.0)
    A2, B2, C2 = _jacobi_coeffs(2)
    A3, B3, C3 = _jacobi_coeffs(3)
    x2_0 = B2 * p - C2
    x2_1 = A2 * p + B2 * q
    x2_2 = A2 * q
    x3_0 = B3 * x2_0 - C3 * p
    x3_1 = A3 * x2_0 + B3 * x2_1 - C3 * q
    x3_2 = A3 * x2_1 + B3 * x2_2
    x3_3 = A3 * x2_2
    c0 = th[0] + th[1] * p + th[2] * x2_0 + th[3] * x3_0
    c1 = th[1] * q + th[2] * x2_1 + th[3] * x3_1
    c2 = th[2] * x2_2 + th[3] * x3_2
    c3 = th[3] * x3_3
    return c0, c1, c2, c3


def _body(th_ref, adj_hbm, out_hbm, inbuf, outbuf, insem, outsem):
    th = _BASETHETA * jnp.tanh(th_ref[0, :])
    c0, c1, c2, c3 = _cubic_coeffs(th)

    def in_copy(step, b):
        return pltpu.make_async_copy(
            adj_hbm.at[pl.ds(step * BR, BR), :], inbuf.at[b], insem.at[b])

    def out_copy(step, b):
        return pltpu.make_async_copy(
            outbuf.at[b], out_hbm.at[pl.ds(step * BR, BR), :], outsem.at[b])

    for b in range(NBUF):
        in_copy(b, b).start()

    def round_step(r, carry):
        for b in range(NBUF):
            step = r * NBUF + b
            in_copy(step, b).wait()

            @pl.when(r > 0)
            def _():
                out_copy(step, b).wait()

            a = inbuf[b]
            outbuf[b] = c0 + a * (c1 + a * (c2 + a * c3))
            out_copy(step, b).start()

            @pl.when(step + NBUF < NSTEPS)
            def _():
                in_copy(step + NBUF, b).start()
        return carry

    lax.fori_loop(0, ROUNDS, round_step, 0)

    for b in range(NBUF):
        out_copy((ROUNDS - 1) * NBUF + b, b).wait()


def kernel(adj, thetas):
    th2d = thetas.reshape(1, _DEPTH + 1)
    return pl.pallas_call(
        _body,
        grid=(),
        in_specs=[
            pl.BlockSpec(memory_space=pltpu.VMEM),
            pl.BlockSpec(memory_space=pl.ANY),
        ],
        out_specs=pl.BlockSpec(memory_space=pl.ANY),
        out_shape=jax.ShapeDtypeStruct((N, N), jnp.float32),
        scratch_shapes=[
            pltpu.VMEM((NBUF, BR, N), jnp.float32),
            pltpu.VMEM((NBUF, BR, N), jnp.float32),
            pltpu.SemaphoreType.DMA((NBUF,)),
            pltpu.SemaphoreType.DMA((NBUF,)),
        ],
    )(th2d, adj)
